# Initial kernel scaffold; baseline (speedup 1.0000x reference)
import jax
import jax.numpy as jnp
from jax.experimental import pallas as pl


def kernel(adj, feats, s):
    raise NotImplementedError("write your pallas kernel here")


# speedup vs baseline: 25.5017x; 25.5017x over previous
import jax
import jax.numpy as jnp
from jax.experimental import pallas as pl
from jax.experimental.pallas import tpu as pltpu


def _prep_kernel(adj_ref, adjb_ref, deg_ref):
    j = pl.program_id(1)
    t = adj_ref[...]
    adjb_ref[...] = t.astype(jnp.bfloat16)
    part = jnp.sum(t, axis=1, keepdims=True)

    @pl.when(j == 0)
    def _init():
        deg_ref[...] = part

    @pl.when(j > 0)
    def _acc():
        deg_ref[...] += part


def _prep(adj, tile=512):
    n = adj.shape[0]
    tile = min(tile, n)
    g = n // tile
    return pl.pallas_call(
        _prep_kernel,
        out_shape=(
            jax.ShapeDtypeStruct((n, n), jnp.bfloat16),
            jax.ShapeDtypeStruct((n, 1), jnp.float32),
        ),
        grid=(g, g),
        in_specs=[pl.BlockSpec((tile, tile), lambda i, j: (i, j))],
        out_specs=(
            pl.BlockSpec((tile, tile), lambda i, j: (i, j)),
            pl.BlockSpec((tile, 1), lambda i, j: (i, 0)),
        ),
        compiler_params=pltpu.CompilerParams(
            dimension_semantics=("parallel", "arbitrary")),
    )(adj)


def _make_fused_kernel(k, n, d, tm):
    nb = n // tm

    def fused(adjb_ref, h0_ref, s_ref, deg_ref, out_ref, h_ref, u_ref):
        s_row = s_ref[...]

        for mi in range(nb):
            rows = pl.ds(mi * tm, tm)
            h0 = h0_ref[rows, :]
            h_ref[rows, :] = h0
            score = jnp.sum(h0 * s_row, axis=1, keepdims=True)
            out_ref[rows, :] = jax.nn.sigmoid(score) * h0

        for _ in range(k):
            for mi in range(nb):
                rows = pl.ds(mi * tm, tm)
                nc = jax.lax.rsqrt(deg_ref[rows, :])
                u_ref[rows, :] = (h_ref[rows, :] * nc).astype(jnp.bfloat16)
            for mi in range(nb):
                rows = pl.ds(mi * tm, tm)
                nc = jax.lax.rsqrt(deg_ref[rows, :])
                y = jnp.dot(adjb_ref[rows, :], u_ref[...],
                            preferred_element_type=jnp.float32)
                h = y * nc
                h_ref[rows, :] = h
                score = jnp.sum(h * s_row, axis=1, keepdims=True)
                out_ref[rows, :] = out_ref[rows, :] + jax.nn.sigmoid(score) * h

    return fused


def _fused_dagnn(adjb, feats, s_row, deg, k, tm=512):
    n, d = feats.shape
    return pl.pallas_call(
        _make_fused_kernel(k, n, d, min(tm, n)),
        out_shape=jax.ShapeDtypeStruct((n, d), jnp.float32),
        grid=(1,),
        in_specs=[
            pl.BlockSpec((n, n), lambda i: (0, 0)),
            pl.BlockSpec((n, d), lambda i: (0, 0)),
            pl.BlockSpec((1, d), lambda i: (0, 0)),
            pl.BlockSpec((n, 1), lambda i: (0, 0)),
        ],
        out_specs=pl.BlockSpec((n, d), lambda i: (0, 0)),
        scratch_shapes=[
            pltpu.VMEM((n, d), jnp.float32),
            pltpu.VMEM((n, d), jnp.bfloat16),
        ],
        compiler_params=pltpu.CompilerParams(
            dimension_semantics=("arbitrary",),
            vmem_limit_bytes=60 * 1024 * 1024),
        cost_estimate=pl.CostEstimate(
            flops=2 * k * n * n * d,
            transcendentals=(k + 1) * n,
            bytes_accessed=2 * n * n + 4 * 3 * n * d),
    )(adjb, feats, s_row, deg)


def kernel(adj, feats, s):
    adjb, deg = _prep(adj.astype(jnp.float32))
    s_row = s.astype(jnp.float32).reshape(1, -1)
    return _fused_dagnn(adjb, feats.astype(jnp.float32), s_row, deg, 4)


# speedup vs baseline: 26.3223x; 1.0322x over previous
import jax
import jax.numpy as jnp
from jax.experimental import pallas as pl
from jax.experimental.pallas import tpu as pltpu


def _prep_kernel(adj_ref, atb_ref, deg_ref):
    j = pl.program_id(1)
    t = adj_ref[...]
    atb_ref[...] = t.T.astype(jnp.bfloat16)
    part = jnp.sum(t, axis=1, keepdims=True)

    @pl.when(j == 0)
    def _init():
        deg_ref[...] = part

    @pl.when(j > 0)
    def _acc():
        deg_ref[...] += part


def _prep(adj, tile=512):
    n = adj.shape[0]
    tile = min(tile, n)
    g = n // tile
    return pl.pallas_call(
        _prep_kernel,
        out_shape=(
            jax.ShapeDtypeStruct((n, n), jnp.bfloat16),
            jax.ShapeDtypeStruct((n, 1), jnp.float32),
        ),
        grid=(g, g),
        in_specs=[pl.BlockSpec((tile, tile), lambda i, j: (i, j))],
        out_specs=(
            pl.BlockSpec((tile, tile), lambda i, j: (j, i)),
            pl.BlockSpec((tile, 1), lambda i, j: (i, 0)),
        ),
        compiler_params=pltpu.CompilerParams(
            dimension_semantics=("parallel", "arbitrary")),
    )(adj)


def _make_fused_kernel(k, n, d, tn):
    nb = n // tn

    def fused(atb_ref, h0t_ref, s_ref, degr_ref, outt_ref, h_ref, u_ref):
        s_col = s_ref[...]

        for ci in range(nb):
            cols = pl.ds(ci * tn, tn)
            h0 = h0t_ref[:, cols]
            h_ref[:, cols] = h0
            score = jnp.sum(h0 * s_col, axis=0, keepdims=True)
            outt_ref[:, cols] = jax.nn.sigmoid(score) * h0

        for _ in range(k):
            for ci in range(nb):
                cols = pl.ds(ci * tn, tn)
                nr = jax.lax.rsqrt(degr_ref[:, cols])
                u_ref[:, cols] = (h_ref[:, cols] * nr).astype(jnp.bfloat16)
            for ci in range(nb):
                cols = pl.ds(ci * tn, tn)
                nr = jax.lax.rsqrt(degr_ref[:, cols])
                y = jnp.dot(u_ref[...], atb_ref[:, cols],
                            preferred_element_type=jnp.float32)
                h = y * nr
                h_ref[:, cols] = h
                score = jnp.sum(h * s_col, axis=0, keepdims=True)
                outt_ref[:, cols] = outt_ref[:, cols] + jax.nn.sigmoid(score) * h

    return fused


def _fused_dagnn(atb, h0t, s, degr, k, tn=512):
    n = atb.shape[0]
    d = h0t.shape[0]
    return pl.pallas_call(
        _make_fused_kernel(k, n, d, min(tn, n)),
        out_shape=jax.ShapeDtypeStruct((d, n), jnp.float32),
        grid=(1,),
        in_specs=[
            pl.BlockSpec((n, n), lambda i: (0, 0)),
            pl.BlockSpec((d, n), lambda i: (0, 0)),
            pl.BlockSpec((d, 1), lambda i: (0, 0)),
            pl.BlockSpec((1, n), lambda i: (0, 0)),
        ],
        out_specs=pl.BlockSpec((d, n), lambda i: (0, 0)),
        scratch_shapes=[
            pltpu.VMEM((d, n), jnp.float32),
            pltpu.VMEM((d, n), jnp.bfloat16),
        ],
        compiler_params=pltpu.CompilerParams(
            dimension_semantics=("arbitrary",),
            vmem_limit_bytes=60 * 1024 * 1024),
        cost_estimate=pl.CostEstimate(
            flops=2 * k * n * n * d,
            transcendentals=(k + 1) * n,
            bytes_accessed=2 * n * n + 4 * 3 * n * d),
    )(atb, h0t, s, degr)


def kernel(adj, feats, s):
    atb, deg = _prep(adj.astype(jnp.float32))
    outt = _fused_dagnn(
        atb,
        feats.astype(jnp.float32).T,
        s.astype(jnp.float32),
        deg.reshape(1, -1),
        4,
    )
    return outt.T


# speedup vs baseline: 36.8548x; 1.4001x over previous
import jax
import jax.numpy as jnp
from jax.experimental import pallas as pl
from jax.experimental.pallas import tpu as pltpu


def _make_prep_kernel(tile, n, tc):
    nc_sub = n // tc

    def prep_kernel(adj_ref, atb_ref, deg_ref):
        t = adj_ref[...]
        deg_ref[...] = jnp.sum(t, axis=1, keepdims=True)
        for c in range(nc_sub):
            atb_ref[pl.ds(c * tc, tc), :] = (
                t[:, c * tc:(c + 1) * tc].T.astype(jnp.bfloat16))

    return prep_kernel


def _prep(adj, tile=512, tc=512):
    n = adj.shape[0]
    tile = min(tile, n)
    tc = min(tc, n)
    g = n // tile
    return pl.pallas_call(
        _make_prep_kernel(tile, n, tc),
        out_shape=(
            jax.ShapeDtypeStruct((n, n), jnp.bfloat16),
            jax.ShapeDtypeStruct((n, 1), jnp.float32),
        ),
        grid=(g,),
        in_specs=[pl.BlockSpec((tile, n), lambda i: (i, 0))],
        out_specs=(
            pl.BlockSpec((n, tile), lambda i: (0, i)),
            pl.BlockSpec((tile, 1), lambda i: (i, 0)),
        ),
        compiler_params=pltpu.CompilerParams(
            dimension_semantics=("parallel",)),
    )(adj)


def _make_fused_kernel(k, n, d, tn):
    nb = n // tn

    def fused(atb_ref, h0t_ref, s_ref, degr_ref, outt_ref, h_ref, u_ref):
        s_col = s_ref[...]

        for ci in range(nb):
            cols = pl.ds(ci * tn, tn)
            h0 = h0t_ref[:, cols]
            h_ref[:, cols] = h0
            score = jnp.sum(h0 * s_col, axis=0, keepdims=True)
            outt_ref[:, cols] = jax.nn.sigmoid(score) * h0

        for _ in range(k):
            for ci in range(nb):
                cols = pl.ds(ci * tn, tn)
                nr = jax.lax.rsqrt(degr_ref[:, cols])
                u_ref[:, cols] = (h_ref[:, cols] * nr).astype(jnp.bfloat16)
            for ci in range(nb):
                cols = pl.ds(ci * tn, tn)
                nr = jax.lax.rsqrt(degr_ref[:, cols])
                y = jnp.dot(u_ref[...], atb_ref[:, cols],
                            preferred_element_type=jnp.float32)
                h = y * nr
                h_ref[:, cols] = h
                score = jnp.sum(h * s_col, axis=0, keepdims=True)
                outt_ref[:, cols] = outt_ref[:, cols] + jax.nn.sigmoid(score) * h

    return fused


def _fused_dagnn(atb, h0t, s, degr, k, tn=512):
    n = atb.shape[0]
    d = h0t.shape[0]
    return pl.pallas_call(
        _make_fused_kernel(k, n, d, min(tn, n)),
        out_shape=jax.ShapeDtypeStruct((d, n), jnp.float32),
        grid=(1,),
        in_specs=[
            pl.BlockSpec((n, n), lambda i: (0, 0)),
            pl.BlockSpec((d, n), lambda i: (0, 0)),
            pl.BlockSpec((d, 1), lambda i: (0, 0)),
            pl.BlockSpec((1, n), lambda i: (0, 0)),
        ],
        out_specs=pl.BlockSpec((d, n), lambda i: (0, 0)),
        scratch_shapes=[
            pltpu.VMEM((d, n), jnp.float32),
            pltpu.VMEM((d, n), jnp.bfloat16),
        ],
        compiler_params=pltpu.CompilerParams(
            dimension_semantics=("arbitrary",),
            vmem_limit_bytes=60 * 1024 * 1024),
        cost_estimate=pl.CostEstimate(
            flops=2 * k * n * n * d,
            transcendentals=(k + 1) * n,
            bytes_accessed=2 * n * n + 4 * 3 * n * d),
    )(atb, h0t, s, degr)


def kernel(adj, feats, s):
    atb, deg = _prep(adj.astype(jnp.float32))
    outt = _fused_dagnn(
        atb,
        feats.astype(jnp.float32).T,
        s.astype(jnp.float32),
        deg.reshape(1, -1),
        4,
    )
    return outt.T


# speedup vs baseline: 54.6534x; 1.4829x over previous
import jax
import jax.numpy as jnp
from jax.experimental import pallas as pl
from jax.experimental.pallas import tpu as pltpu


def _make_merged_kernel(k, n, d, tile, tn):
    ns = n // tile
    nb = n // tn

    def body(adj_ref, h0_ref, s_ref, out_ref,
             atb_ref, degr_ref, h_ref, u_ref, outt_ref):
        i = pl.program_id(0)

        @pl.when(i < ns)
        def _phase1():
            t = adj_ref[...]
            acc = jnp.zeros((1, tile), jnp.float32)
            for c in range(nb):
                ttf = t[:, c * tn:(c + 1) * tn].T
                atb_ref[pl.ds(c * tn, tn), pl.ds(i * tile, tile)] = (
                    ttf.astype(jnp.bfloat16))
                acc = acc + jnp.sum(ttf, axis=0, keepdims=True)
            degr_ref[:, pl.ds(i * tile, tile)] = acc

        @pl.when(i == ns)
        def _phase2():
            s_col = s_ref[...]

            for ci in range(nb):
                cols = pl.ds(ci * tn, tn)
                ft = h0_ref[pl.ds(ci * tn, tn), :].T
                h_ref[:, cols] = ft
                score = jnp.sum(ft * s_col, axis=0, keepdims=True)
                outt_ref[:, cols] = jax.nn.sigmoid(score) * ft

            for _ in range(k):
                for ci in range(nb):
                    cols = pl.ds(ci * tn, tn)
                    nr = jax.lax.rsqrt(degr_ref[:, cols])
                    u_ref[:, cols] = (h_ref[:, cols] * nr).astype(jnp.bfloat16)
                for ci in range(nb):
                    cols = pl.ds(ci * tn, tn)
                    nr = jax.lax.rsqrt(degr_ref[:, cols])
                    y = jnp.dot(u_ref[...], atb_ref[:, cols],
                                preferred_element_type=jnp.float32)
                    h = y * nr
                    h_ref[:, cols] = h
                    score = jnp.sum(h * s_col, axis=0, keepdims=True)
                    outt_ref[:, cols] = (outt_ref[:, cols]
                                         + jax.nn.sigmoid(score) * h)

            for ci in range(nb):
                out_ref[pl.ds(ci * tn, tn), :] = outt_ref[:, pl.ds(ci * tn, tn)].T

    return body


def _dagnn(adj, feats, s, k, tile=256, tn=512):
    n, d = feats.shape
    tile = min(tile, n)
    tn = min(tn, n)
    ns = n // tile
    return pl.pallas_call(
        _make_merged_kernel(k, n, d, tile, tn),
        out_shape=jax.ShapeDtypeStruct((n, d), jnp.float32),
        grid=(ns + 1,),
        in_specs=[
            pl.BlockSpec((tile, n), lambda i: (jnp.minimum(i, ns - 1), 0)),
            pl.BlockSpec((n, d), lambda i: (0, 0)),
            pl.BlockSpec((d, 1), lambda i: (0, 0)),
        ],
        out_specs=pl.BlockSpec((n, d), lambda i: (0, 0)),
        scratch_shapes=[
            pltpu.VMEM((n, n), jnp.bfloat16),
            pltpu.VMEM((1, n), jnp.float32),
            pltpu.VMEM((d, n), jnp.float32),
            pltpu.VMEM((d, n), jnp.bfloat16),
            pltpu.VMEM((d, n), jnp.float32),
        ],
        compiler_params=pltpu.CompilerParams(
            dimension_semantics=("arbitrary",),
            vmem_limit_bytes=60 * 1024 * 1024),
        cost_estimate=pl.CostEstimate(
            flops=2 * k * n * n * d,
            transcendentals=(k + 1) * n,
            bytes_accessed=4 * n * n + 4 * 3 * n * d),
    )(adj.astype(jnp.float32), feats.astype(jnp.float32), s.astype(jnp.float32))


def kernel(adj, feats, s):
    return _dagnn(adj, feats, s, 4)


# speedup vs baseline: 58.3620x; 1.0679x over previous
import jax
import jax.numpy as jnp
from jax.experimental import pallas as pl
from jax.experimental.pallas import tpu as pltpu


def _make_merged_kernel(k, n, d, tile, tn):
    ns = n // tile
    nb = n // tn

    def body(adj_ref, h0_ref, s_ref, out_ref,
             atb_ref, degr_ref, h_ref, u_ref, outt_ref):
        i = pl.program_id(0)

        @pl.when(i < ns)
        def _phase1():
            t = adj_ref[...]
            acc = jnp.zeros((1, tile), jnp.float32)
            for c in range(nb):
                ttf = t[:, c * tn:(c + 1) * tn].T
                atb_ref[pl.ds(c * tn, tn), pl.ds(i * tile, tile)] = (
                    ttf.astype(jnp.bfloat16))
                acc = acc + jnp.sum(ttf, axis=0, keepdims=True)
            degr_ref[:, pl.ds(i * tile, tile)] = acc

        @pl.when(i == ns)
        def _phase2():
            s_col = s_ref[...]

            for ci in range(nb):
                cols = pl.ds(ci * tn, tn)
                ft = h0_ref[pl.ds(ci * tn, tn), :].T
                h_ref[:, cols] = ft
                score = jnp.sum(ft * s_col, axis=0, keepdims=True)
                outt_ref[:, cols] = jax.nn.sigmoid(score) * ft

            for _ in range(k):
                for ci in range(nb):
                    cols = pl.ds(ci * tn, tn)
                    nr = jax.lax.rsqrt(degr_ref[:, cols])
                    u_ref[:, cols] = (h_ref[:, cols] * nr).astype(jnp.bfloat16)
                for ci in range(nb):
                    cols = pl.ds(ci * tn, tn)
                    nr = jax.lax.rsqrt(degr_ref[:, cols])
                    y = jnp.dot(u_ref[...], atb_ref[:, cols],
                                preferred_element_type=jnp.float32)
                    h = y * nr
                    h_ref[:, cols] = h
                    score = jnp.sum(h * s_col, axis=0, keepdims=True)
                    outt_ref[:, cols] = (outt_ref[:, cols]
                                         + jax.nn.sigmoid(score) * h)

            for ci in range(nb):
                out_ref[pl.ds(ci * tn, tn), :] = outt_ref[:, pl.ds(ci * tn, tn)].T

    return body


def _dagnn(adj, feats, s, k, tile=512, tn=512):
    n, d = feats.shape
    tile = min(tile, n)
    tn = min(tn, n)
    ns = n // tile
    return pl.pallas_call(
        _make_merged_kernel(k, n, d, tile, tn),
        out_shape=jax.ShapeDtypeStruct((n, d), jnp.float32),
        grid=(ns + 1,),
        in_specs=[
            pl.BlockSpec((tile, n), lambda i: (jnp.minimum(i, ns - 1), 0)),
            pl.BlockSpec((n, d), lambda i: (0, 0)),
            pl.BlockSpec((d, 1), lambda i: (0, 0)),
        ],
        out_specs=pl.BlockSpec((n, d), lambda i: (0, 0)),
        scratch_shapes=[
            pltpu.VMEM((n, n), jnp.bfloat16),
            pltpu.VMEM((1, n), jnp.float32),
            pltpu.VMEM((d, n), jnp.float32),
            pltpu.VMEM((d, n), jnp.bfloat16),
            pltpu.VMEM((d, n), jnp.float32),
        ],
        compiler_params=pltpu.CompilerParams(
            dimension_semantics=("arbitrary",),
            vmem_limit_bytes=63 * 1024 * 1024),
        cost_estimate=pl.CostEstimate(
            flops=2 * k * n * n * d,
            transcendentals=(k + 1) * n,
            bytes_accessed=4 * n * n + 4 * 3 * n * d),
    )(adj.astype(jnp.float32), feats.astype(jnp.float32), s.astype(jnp.float32))


def kernel(adj, feats, s):
    return _dagnn(adj, feats, s, 4)


# speedup vs baseline: 58.6497x; 1.0049x over previous
import jax
import jax.numpy as jnp
from jax.experimental import pallas as pl
from jax.experimental.pallas import tpu as pltpu


def _make_merged_kernel(k, n, d, tile, tn):
    ns = n // tile
    nb = n // tn

    half = tile // 2

    def body(adj0_ref, adj1_ref, h0_ref, s_ref, out_ref,
             atb_ref, degr_ref, h_ref, u_ref, outt_ref):
        i = pl.program_id(0)

        @pl.when(i < ns)
        def _phase1():
            for hh, aref in enumerate((adj0_ref, adj1_ref)):
                t = aref[...]
                acc = jnp.zeros((1, half), jnp.float32)
                for c in range(nb):
                    ttf = t[:, c * tn:(c + 1) * tn].T
                    atb_ref[pl.ds(c * tn, tn),
                            pl.ds(i * tile + hh * half, half)] = (
                        ttf.astype(jnp.bfloat16))
                    acc = acc + jnp.sum(ttf, axis=0, keepdims=True)
                degr_ref[:, pl.ds(i * tile + hh * half, half)] = acc

        @pl.when(i == ns)
        def _phase2():
            s_col = s_ref[...]

            for ci in range(nb):
                cols = pl.ds(ci * tn, tn)
                ft = h0_ref[pl.ds(ci * tn, tn), :].T
                h_ref[:, cols] = ft
                score = jnp.sum(ft * s_col, axis=0, keepdims=True)
                outt_ref[:, cols] = jax.nn.sigmoid(score) * ft

            for _ in range(k):
                for ci in range(nb):
                    cols = pl.ds(ci * tn, tn)
                    nr = jax.lax.rsqrt(degr_ref[:, cols])
                    u_ref[:, cols] = (h_ref[:, cols] * nr).astype(jnp.bfloat16)
                for ci in range(nb):
                    cols = pl.ds(ci * tn, tn)
                    nr = jax.lax.rsqrt(degr_ref[:, cols])
                    y = jnp.dot(u_ref[...], atb_ref[:, cols],
                                preferred_element_type=jnp.float32)
                    h = y * nr
                    h_ref[:, cols] = h
                    score = jnp.sum(h * s_col, axis=0, keepdims=True)
                    outt_ref[:, cols] = (outt_ref[:, cols]
                                         + jax.nn.sigmoid(score) * h)

            for ci in range(nb):
                out_ref[pl.ds(ci * tn, tn), :] = outt_ref[:, pl.ds(ci * tn, tn)].T

    return body


def _dagnn(adj, feats, s, k, tile=512, tn=512):
    n, d = feats.shape
    tile = min(tile, n)
    tn = min(tn, n)
    ns = n // tile
    return pl.pallas_call(
        _make_merged_kernel(k, n, d, tile, tn),
        out_shape=jax.ShapeDtypeStruct((n, d), jnp.float32),
        grid=(ns + 1,),
        in_specs=[
            pl.BlockSpec((tile // 2, n),
                         lambda i: (2 * jnp.minimum(i, ns - 1), 0)),
            pl.BlockSpec((tile // 2, n),
                         lambda i: (2 * jnp.minimum(i, ns - 1) + 1, 0)),
            pl.BlockSpec((n, d), lambda i: (0, 0)),
            pl.BlockSpec((d, 1), lambda i: (0, 0)),
        ],
        out_specs=pl.BlockSpec((n, d), lambda i: (0, 0)),
        scratch_shapes=[
            pltpu.VMEM((n, n), jnp.bfloat16),
            pltpu.VMEM((1, n), jnp.float32),
            pltpu.VMEM((d, n), jnp.float32),
            pltpu.VMEM((d, n), jnp.bfloat16),
            pltpu.VMEM((d, n), jnp.float32),
        ],
        compiler_params=pltpu.CompilerParams(
            dimension_semantics=("arbitrary",),
            vmem_limit_bytes=63 * 1024 * 1024),
        cost_estimate=pl.CostEstimate(
            flops=2 * k * n * n * d,
            transcendentals=(k + 1) * n,
            bytes_accessed=4 * n * n + 4 * 3 * n * d),
    )(adj.astype(jnp.float32), adj.astype(jnp.float32),
      feats.astype(jnp.float32), s.astype(jnp.float32))


def kernel(adj, feats, s):
    return _dagnn(adj, feats, s, 4)
